# Initial kernel scaffold; baseline (speedup 1.0000x reference)
import jax
import jax.numpy as jnp
from jax.experimental import pallas as pl


def kernel(x, p1, W, b, gamma, beta):
    raise NotImplementedError("write your pallas kernel here")


# speedup vs baseline: 12.3279x; 12.3279x over previous
import functools

import jax
import jax.numpy as jnp
from jax import lax
from jax.experimental import pallas as pl
from jax.experimental.pallas import tpu as pltpu
from jax.experimental.pallas import tpu_sc as plsc

_N = 16384
_M = 4096
_K = 16
_R = 128
_C = 128
_QT = 128


def _fps_body(px_ref, py_ref, pz_ref, p2_ref):
    px = px_ref[...]
    py = py_ref[...]
    pz = pz_ref[...]
    iota_lin = (lax.broadcasted_iota(jnp.int32, (_R, _C), 0) * _C
                + lax.broadcasted_iota(jnp.int32, (_R, _C), 1))
    lane = lax.broadcasted_iota(jnp.int32, (1, _C), 1)

    def coords_at(r, c):
        rowx = px_ref[pl.ds(r, 1), :]
        rowy = py_ref[pl.ds(r, 1), :]
        rowz = pz_ref[pl.ds(r, 1), :]
        sel = lane == c
        cx = jnp.sum(jnp.where(sel, rowx, 0.0))
        cy = jnp.sum(jnp.where(sel, rowy, 0.0))
        cz = jnp.sum(jnp.where(sel, rowz, 0.0))
        return cx, cy, cz

    def store_row(i, cx, cy, cz):
        row = jnp.where(lane == 0, cx,
                        jnp.where(lane == 1, cy,
                                  jnp.where(lane == 2, cz, 0.0)))
        p2_ref[pl.ds(i, 1), :] = row

    lx, ly, lz = coords_at(0, 0)
    store_row(0, lx, ly, lz)

    def body(i, state):
        dist, cx, cy, cz = state
        dx = px - cx
        dy = py - cy
        dz = pz - cz
        d = dx * dx + dy * dy + dz * dz
        dist = jnp.minimum(dist, d)
        m = jnp.max(dist)
        sel = jnp.where(dist == m, iota_lin, _N)
        j = jnp.min(sel)
        r = j // _C
        c = j - r * _C
        nx, ny, nz = coords_at(r, c)
        store_row(i, nx, ny, nz)
        return dist, nx, ny, nz

    dist0 = jnp.full((_R, _C), jnp.inf, dtype=jnp.float32)
    lax.fori_loop(1, _M, body, (dist0, lx, ly, lz), unroll=False)


def _fps(pxg, pyg, pzg):
    return pl.pallas_call(
        _fps_body,
        out_shape=jax.ShapeDtypeStruct((_M, 128), jnp.float32),
    )(pxg, pyg, pzg)


def _knn_body(p2_ref, pxr_ref, pyr_ref, pzr_ref, out_ref):
    qx = p2_ref[:, 0:1]
    qy = p2_ref[:, 1:2]
    qz = p2_ref[:, 2:3]
    rx = pxr_ref[...]
    ry = pyr_ref[...]
    rz = pzr_ref[...]
    bf = lambda v: v.astype(jnp.bfloat16).astype(jnp.float32)
    qxb, qyb, qzb = bf(qx), bf(qy), bf(qz)
    rxb, ryb, rzb = bf(rx), bf(ry), bf(rz)
    dot = (qxb * rxb + qyb * ryb) + qzb * rzb
    sqq = (qx * qx + qy * qy) + qz * qz
    sqr = (rx * rx + ry * ry) + rz * rz
    d = (sqq - 2.0 * dot) + sqr
    idx_row = lax.broadcasted_iota(jnp.int32, (_QT, _N), 1)
    lane = lax.broadcasted_iota(jnp.int32, (_QT, 128), 1)
    out = jnp.zeros((_QT, 128), dtype=jnp.int32)
    for k in range(_K):
        m = jnp.min(d, axis=1, keepdims=True)
        sel = jnp.where(d == m, idx_row, _N)
        j = jnp.min(sel, axis=1, keepdims=True)
        out = jnp.where(lane == k, j, out)
        d = jnp.where(idx_row == j, jnp.inf, d)
    out_ref[...] = out


def _knn(p2pad, pxr, pyr, pzr):
    grid = _M // _QT
    return pl.pallas_call(
        _knn_body,
        grid=(grid,),
        in_specs=[
            pl.BlockSpec((_QT, 128), lambda i: (i, 0)),
            pl.BlockSpec((1, _N), lambda i: (0, 0)),
            pl.BlockSpec((1, _N), lambda i: (0, 0)),
            pl.BlockSpec((1, _N), lambda i: (0, 0)),
        ],
        out_specs=pl.BlockSpec((_QT, 128), lambda i: (i, 0)),
        out_shape=jax.ShapeDtypeStruct((_M, 128), jnp.int32),
    )(p2pad, pxr, pyr, pzr)


def _mlp_body(x_ref, w_ref, b_ref, g_ref, beta_ref, h_ref):
    h = jnp.dot(x_ref[...], w_ref[...],
                preferred_element_type=jnp.float32) + b_ref[...]
    mean = jnp.mean(h, axis=0, keepdims=True)
    ctr = h - mean
    var = jnp.mean(ctr * ctr, axis=0, keepdims=True)
    scale = g_ref[...] * lax.rsqrt(var + 1e-5)
    h_ref[...] = jnp.maximum(ctr * scale + beta_ref[...], 0.0)


def _mlp(x, W, b2, g2, beta2):
    return pl.pallas_call(
        _mlp_body,
        out_shape=jax.ShapeDtypeStruct((_N, 128), jnp.float32),
    )(x, W, b2, g2, beta2)


_NC = 2
_NS = 16
_NW = _NC * _NS
_QPW = _M // _NW
_QCH = 8
_ICH = _QCH * _K
_NCH = _QPW // _QCH


def _sc_gather_body(h_hbm, idx_hbm, y_hbm, idx_v, rows_v, out_v, sem):
    wid = lax.axis_index("s") * _NC + lax.axis_index("c")
    base = wid * (_QPW * _K)

    def chunk_body(ch, carry):
        pltpu.sync_copy(idx_hbm.at[pl.ds(base + ch * _ICH, _ICH)], idx_v)
        pltpu.async_copy(h_hbm.at[idx_v], rows_v, sem).wait()

        def q_body(q, carry2):
            for col in range(8):
                sl = pl.ds(col * 16, 16)
                acc = rows_v[q * _K, sl]
                for r in range(1, _K):
                    acc = jnp.maximum(acc, rows_v[q * _K + r, sl])
                out_v[ch * _QCH + q, sl] = acc
            return carry2

        lax.fori_loop(0, _QCH, q_body, 0, unroll=False)
        return carry

    lax.fori_loop(0, _NCH, chunk_body, 0, unroll=False)
    pltpu.sync_copy(out_v, y_hbm.at[pl.ds(wid * _QPW, _QPW)])


def _sc_gather_max(h, idx_flat):
    mesh = plsc.VectorSubcoreMesh(core_axis_name="c", subcore_axis_name="s")
    f = pl.kernel(
        _sc_gather_body,
        out_type=jax.ShapeDtypeStruct((_M, 128), jnp.float32),
        mesh=mesh,
        scratch_types=[
            pltpu.VMEM((_ICH,), jnp.int32),
            pltpu.VMEM((_ICH, 128), jnp.float32),
            pltpu.VMEM((_QPW, 128), jnp.float32),
            pltpu.SemaphoreType.DMA,
        ],
    )
    return f(h, idx_flat)


@jax.jit
def kernel(x, p1, W, b, gamma, beta):
    pxg = p1[:, 0].reshape(_R, _C)
    pyg = p1[:, 1].reshape(_R, _C)
    pzg = p1[:, 2].reshape(_R, _C)
    p2pad = _fps(pxg, pyg, pzg)

    pxr = p1[:, 0].reshape(1, _N)
    pyr = p1[:, 1].reshape(1, _N)
    pzr = p1[:, 2].reshape(1, _N)
    nbrs = _knn(p2pad, pxr, pyr, pzr)

    h = _mlp(x, W, b.reshape(1, 128), gamma.reshape(1, 128),
             beta.reshape(1, 128))

    idx_flat = nbrs[:, :_K].reshape(_M * _K)
    y = _sc_gather_max(h, idx_flat)

    p2 = p2pad[:, :3]
    return (y, p2)


# speedup vs baseline: 12.6788x; 1.0285x over previous
import functools

import jax
import jax.numpy as jnp
from jax import lax
from jax.experimental import pallas as pl
from jax.experimental.pallas import tpu as pltpu
from jax.experimental.pallas import tpu_sc as plsc

_N = 16384
_M = 4096
_K = 16
_R = 128
_C = 128
_QT = 128


def _fps_body(px_ref, py_ref, pz_ref, p2_ref):
    px = px_ref[...]
    py = py_ref[...]
    pz = pz_ref[...]
    lane = lax.broadcasted_iota(jnp.int32, (1, _C), 1)

    def coords_at(r, c):
        rowx = px_ref[pl.ds(r, 1), :]
        rowy = py_ref[pl.ds(r, 1), :]
        rowz = pz_ref[pl.ds(r, 1), :]
        sel = lane == c
        cx = jnp.sum(jnp.where(sel, rowx, 0.0))
        cy = jnp.sum(jnp.where(sel, rowy, 0.0))
        cz = jnp.sum(jnp.where(sel, rowz, 0.0))
        return cx, cy, cz

    def store_row(i, cx, cy, cz):
        row = jnp.where(lane == 0, cx,
                        jnp.where(lane == 1, cy,
                                  jnp.where(lane == 2, cz, 0.0)))
        p2_ref[pl.ds(i, 1), :] = row

    lx, ly, lz = coords_at(0, 0)
    store_row(0, lx, ly, lz)

    def body(i, state):
        dist, cx, cy, cz = state
        dx = px - cx
        dy = py - cy
        dz = pz - cz
        d = dx * dx + dy * dy + dz * dz
        dist = jnp.minimum(dist, d)
        j = jnp.argmax(dist).astype(jnp.int32)
        r = j // _C
        c = j - r * _C
        nx, ny, nz = coords_at(r, c)
        store_row(i, nx, ny, nz)
        return dist, nx, ny, nz

    dist0 = jnp.full((_R, _C), jnp.inf, dtype=jnp.float32)
    lax.fori_loop(1, _M, body, (dist0, lx, ly, lz), unroll=False)


def _fps(pxg, pyg, pzg):
    return pl.pallas_call(
        _fps_body,
        out_shape=jax.ShapeDtypeStruct((_M, 128), jnp.float32),
    )(pxg, pyg, pzg)


def _knn_body(p2_ref, pxr_ref, pyr_ref, pzr_ref, out_ref):
    qx = p2_ref[:, 0:1]
    qy = p2_ref[:, 1:2]
    qz = p2_ref[:, 2:3]
    rx = pxr_ref[...]
    ry = pyr_ref[...]
    rz = pzr_ref[...]
    bf = lambda v: v.astype(jnp.bfloat16).astype(jnp.float32)
    qxb, qyb, qzb = bf(qx), bf(qy), bf(qz)
    rxb, ryb, rzb = bf(rx), bf(ry), bf(rz)
    dot = (qxb * rxb + qyb * ryb) + qzb * rzb
    sqq = (qx * qx + qy * qy) + qz * qz
    sqr = (rx * rx + ry * ry) + rz * rz
    d = (sqq - 2.0 * dot) + sqr
    idx_row = lax.broadcasted_iota(jnp.int32, (_QT, _N), 1)
    lane = lax.broadcasted_iota(jnp.int32, (_QT, 128), 1)
    out = jnp.zeros((_QT, 128), dtype=jnp.int32)
    for k in range(_K):
        j = jnp.argmin(d, axis=1).astype(jnp.int32).reshape(_QT, 1)
        out = jnp.where(lane == k, j, out)
        d = jnp.where(idx_row == j, jnp.inf, d)
    out_ref[...] = out


def _knn(p2pad, pxr, pyr, pzr):
    grid = _M // _QT
    return pl.pallas_call(
        _knn_body,
        grid=(grid,),
        in_specs=[
            pl.BlockSpec((_QT, 128), lambda i: (i, 0)),
            pl.BlockSpec((1, _N), lambda i: (0, 0)),
            pl.BlockSpec((1, _N), lambda i: (0, 0)),
            pl.BlockSpec((1, _N), lambda i: (0, 0)),
        ],
        out_specs=pl.BlockSpec((_QT, 128), lambda i: (i, 0)),
        out_shape=jax.ShapeDtypeStruct((_M, 128), jnp.int32),
    )(p2pad, pxr, pyr, pzr)


def _mlp_body(x_ref, w_ref, b_ref, g_ref, beta_ref, h_ref):
    h = jnp.dot(x_ref[...], w_ref[...],
                preferred_element_type=jnp.float32) + b_ref[...]
    mean = jnp.mean(h, axis=0, keepdims=True)
    ctr = h - mean
    var = jnp.mean(ctr * ctr, axis=0, keepdims=True)
    scale = g_ref[...] * lax.rsqrt(var + 1e-5)
    h_ref[...] = jnp.maximum(ctr * scale + beta_ref[...], 0.0)


def _mlp(x, W, b2, g2, beta2):
    return pl.pallas_call(
        _mlp_body,
        out_shape=jax.ShapeDtypeStruct((_N, 128), jnp.float32),
    )(x, W, b2, g2, beta2)


_NC = 2
_NS = 16
_NW = _NC * _NS
_QPW = _M // _NW
_QCH = 8
_ICH = _QCH * _K
_NCH = _QPW // _QCH


def _sc_gather_body(h_hbm, idx_hbm, y_hbm, idx_v, rows_v, out_v, sem):
    wid = lax.axis_index("s") * _NC + lax.axis_index("c")
    base = wid * (_QPW * _K)

    def chunk_body(ch, carry):
        pltpu.sync_copy(idx_hbm.at[pl.ds(base + ch * _ICH, _ICH)], idx_v)
        pltpu.async_copy(h_hbm.at[idx_v], rows_v, sem).wait()

        def q_body(q, carry2):
            for col in range(8):
                sl = pl.ds(col * 16, 16)
                acc = rows_v[q * _K, sl]
                for r in range(1, _K):
                    acc = jnp.maximum(acc, rows_v[q * _K + r, sl])
                out_v[ch * _QCH + q, sl] = acc
            return carry2

        lax.fori_loop(0, _QCH, q_body, 0, unroll=False)
        return carry

    lax.fori_loop(0, _NCH, chunk_body, 0, unroll=False)
    pltpu.sync_copy(out_v, y_hbm.at[pl.ds(wid * _QPW, _QPW)])


def _sc_gather_max(h, idx_flat):
    mesh = plsc.VectorSubcoreMesh(core_axis_name="c", subcore_axis_name="s")
    f = pl.kernel(
        _sc_gather_body,
        out_type=jax.ShapeDtypeStruct((_M, 128), jnp.float32),
        mesh=mesh,
        scratch_types=[
            pltpu.VMEM((_ICH,), jnp.int32),
            pltpu.VMEM((_ICH, 128), jnp.float32),
            pltpu.VMEM((_QPW, 128), jnp.float32),
            pltpu.SemaphoreType.DMA,
        ],
    )
    return f(h, idx_flat)


@jax.jit
def kernel(x, p1, W, b, gamma, beta):
    pxg = p1[:, 0].reshape(_R, _C)
    pyg = p1[:, 1].reshape(_R, _C)
    pzg = p1[:, 2].reshape(_R, _C)
    p2pad = _fps(pxg, pyg, pzg)

    pxr = p1[:, 0].reshape(1, _N)
    pyr = p1[:, 1].reshape(1, _N)
    pzr = p1[:, 2].reshape(1, _N)
    nbrs = _knn(p2pad, pxr, pyr, pzr)

    h = _mlp(x, W, b.reshape(1, 128), gamma.reshape(1, 128),
             beta.reshape(1, 128))

    idx_flat = nbrs[:, :_K].reshape(_M * _K)
    y = _sc_gather_max(h, idx_flat)

    p2 = p2pad[:, :3]
    return (y, p2)


# speedup vs baseline: 14.6411x; 1.1548x over previous
import functools

import jax
import jax.numpy as jnp
from jax import lax
from jax.experimental import pallas as pl
from jax.experimental.pallas import tpu as pltpu
from jax.experimental.pallas import tpu_sc as plsc

_N = 16384
_M = 4096
_K = 16
_R = 128
_C = 128
_QT = 128


def _fps_body(px_ref, py_ref, pz_ref, pxs_ref, pys_ref, pzs_ref, p2_ref):
    px = px_ref[...]
    py = py_ref[...]
    pz = pz_ref[...]
    lane = lax.broadcasted_iota(jnp.int32, (1, _C), 1)

    def store_row(i, cx, cy, cz):
        row = jnp.where(lane == 0, cx,
                        jnp.where(lane == 1, cy,
                                  jnp.where(lane == 2, cz, 0.0)))
        p2_ref[pl.ds(i, 1), :] = row

    lx, ly, lz = pxs_ref[0], pys_ref[0], pzs_ref[0]
    store_row(0, lx, ly, lz)

    def body(i, state):
        dist, cx, cy, cz = state
        dx = px - cx
        dy = py - cy
        dz = pz - cz
        d = dx * dx + dy * dy + dz * dz
        dist = jnp.minimum(dist, d)
        j = jnp.argmax(dist).astype(jnp.int32)
        nx, ny, nz = pxs_ref[j], pys_ref[j], pzs_ref[j]
        store_row(i, nx, ny, nz)
        return dist, nx, ny, nz

    dist0 = jnp.full((_R, _C), jnp.inf, dtype=jnp.float32)
    lax.fori_loop(1, _M, body, (dist0, lx, ly, lz), unroll=False)


def _fps(pxg, pyg, pzg, pxs, pys, pzs):
    return pl.pallas_call(
        _fps_body,
        in_specs=[pl.BlockSpec(memory_space=pltpu.VMEM)] * 3
                 + [pl.BlockSpec(memory_space=pltpu.SMEM)] * 3,
        out_shape=jax.ShapeDtypeStruct((_M, 128), jnp.float32),
    )(pxg, pyg, pzg, pxs, pys, pzs)


def _knn_body(p2_ref, pxr_ref, pyr_ref, pzr_ref, out_ref):
    qx = p2_ref[:, 0:1]
    qy = p2_ref[:, 1:2]
    qz = p2_ref[:, 2:3]
    rx = pxr_ref[...]
    ry = pyr_ref[...]
    rz = pzr_ref[...]
    bf = lambda v: v.astype(jnp.bfloat16).astype(jnp.float32)
    qxb, qyb, qzb = bf(qx), bf(qy), bf(qz)
    rxb, ryb, rzb = bf(rx), bf(ry), bf(rz)
    dot = (qxb * rxb + qyb * ryb) + qzb * rzb
    sqq = (qx * qx + qy * qy) + qz * qz
    sqr = (rx * rx + ry * ry) + rz * rz
    d = (sqq - 2.0 * dot) + sqr
    idx_row = lax.broadcasted_iota(jnp.int32, (_QT, _N), 1)
    lane = lax.broadcasted_iota(jnp.int32, (_QT, 128), 1)
    out = jnp.zeros((_QT, 128), dtype=jnp.int32)
    for k in range(_K):
        j = jnp.argmin(d, axis=1).astype(jnp.int32).reshape(_QT, 1)
        out = jnp.where(lane == k, j, out)
        d = jnp.where(idx_row == j, jnp.inf, d)
    out_ref[...] = out


def _knn(p2pad, pxr, pyr, pzr):
    grid = _M // _QT
    return pl.pallas_call(
        _knn_body,
        grid=(grid,),
        in_specs=[
            pl.BlockSpec((_QT, 128), lambda i: (i, 0)),
            pl.BlockSpec((1, _N), lambda i: (0, 0)),
            pl.BlockSpec((1, _N), lambda i: (0, 0)),
            pl.BlockSpec((1, _N), lambda i: (0, 0)),
        ],
        out_specs=pl.BlockSpec((_QT, 128), lambda i: (i, 0)),
        out_shape=jax.ShapeDtypeStruct((_M, 128), jnp.int32),
    )(p2pad, pxr, pyr, pzr)


def _mlp_body(x_ref, w_ref, b_ref, g_ref, beta_ref, h_ref):
    h = jnp.dot(x_ref[...], w_ref[...],
                preferred_element_type=jnp.float32) + b_ref[...]
    mean = jnp.mean(h, axis=0, keepdims=True)
    ctr = h - mean
    var = jnp.mean(ctr * ctr, axis=0, keepdims=True)
    scale = g_ref[...] * lax.rsqrt(var + 1e-5)
    h_ref[...] = jnp.maximum(ctr * scale + beta_ref[...], 0.0)


def _mlp(x, W, b2, g2, beta2):
    return pl.pallas_call(
        _mlp_body,
        out_shape=jax.ShapeDtypeStruct((_N, 128), jnp.float32),
    )(x, W, b2, g2, beta2)


_NC = 2
_NS = 16
_NW = _NC * _NS
_QPW = _M // _NW
_QCH = 8
_ICH = _QCH * _K
_NCH = _QPW // _QCH


def _sc_gather_body(h_hbm, idx_hbm, y_hbm, idx_v, rows_v, out_v, sem):
    wid = lax.axis_index("s") * _NC + lax.axis_index("c")
    base = wid * (_QPW * _K)

    def chunk_body(ch, carry):
        pltpu.sync_copy(idx_hbm.at[pl.ds(base + ch * _ICH, _ICH)], idx_v)
        pltpu.async_copy(h_hbm.at[idx_v], rows_v, sem).wait()

        def q_body(q, carry2):
            for col in range(8):
                sl = pl.ds(col * 16, 16)
                acc = rows_v[q * _K, sl]
                for r in range(1, _K):
                    acc = jnp.maximum(acc, rows_v[q * _K + r, sl])
                out_v[ch * _QCH + q, sl] = acc
            return carry2

        lax.fori_loop(0, _QCH, q_body, 0, unroll=False)
        return carry

    lax.fori_loop(0, _NCH, chunk_body, 0, unroll=False)
    pltpu.sync_copy(out_v, y_hbm.at[pl.ds(wid * _QPW, _QPW)])


def _sc_gather_max(h, idx_flat):
    mesh = plsc.VectorSubcoreMesh(core_axis_name="c", subcore_axis_name="s")
    f = pl.kernel(
        _sc_gather_body,
        out_type=jax.ShapeDtypeStruct((_M, 128), jnp.float32),
        mesh=mesh,
        scratch_types=[
            pltpu.VMEM((_ICH,), jnp.int32),
            pltpu.VMEM((_ICH, 128), jnp.float32),
            pltpu.VMEM((_QPW, 128), jnp.float32),
            pltpu.SemaphoreType.DMA,
        ],
    )
    return f(h, idx_flat)


@jax.jit
def kernel(x, p1, W, b, gamma, beta):
    pxs = p1[:, 0]
    pys = p1[:, 1]
    pzs = p1[:, 2]
    pxg = pxs.reshape(_R, _C)
    pyg = pys.reshape(_R, _C)
    pzg = pzs.reshape(_R, _C)
    p2pad = _fps(pxg, pyg, pzg, pxs, pys, pzs)

    pxr = p1[:, 0].reshape(1, _N)
    pyr = p1[:, 1].reshape(1, _N)
    pzr = p1[:, 2].reshape(1, _N)
    nbrs = _knn(p2pad, pxr, pyr, pzr)

    h = _mlp(x, W, b.reshape(1, 128), gamma.reshape(1, 128),
             beta.reshape(1, 128))

    idx_flat = nbrs[:, :_K].reshape(_M * _K)
    y = _sc_gather_max(h, idx_flat)

    p2 = p2pad[:, :3]
    return (y, p2)
